# baseline (speedup 1.0000x reference)
import functools

import jax
import jax.numpy as jnp
from jax import lax
from jax.experimental import pallas as pl
from jax.experimental.pallas import tpu as pltpu
from jax.experimental.pallas import tpu_sc as plsc

_NC = 2
_NS = 16
_NW = _NC * _NS


def _gather_rows(table, idx, chunk, k_streams):
    (n,) = idx.shape
    _, d = table.shape
    b_per_w = n // _NW
    group = chunk * k_streams
    n_super = b_per_w // group
    nslots = 2 * k_streams
    mesh = plsc.VectorSubcoreMesh(core_axis_name="c", subcore_axis_name="s")

    @functools.partial(
        pl.kernel,
        out_type=jax.ShapeDtypeStruct((n, d), table.dtype),
        mesh=mesh,
        scratch_types=[
            pltpu.VMEM((nslots, chunk), jnp.int32),
            pltpu.VMEM((nslots, chunk, d), jnp.float32),
            pltpu.SemaphoreType.DMA,
            pltpu.SemaphoreType.DMA,
            pltpu.SemaphoreType.DMA,
        ],
        compiler_params=pltpu.CompilerParams(use_tc_tiling_on_sc=False),
    )
    def k(table_hbm, idx_hbm, out_hbm, idx_v, rows_v, isem, gsem, osem):
        wid = lax.axis_index("s") * _NC + lax.axis_index("c")
        base = wid * b_per_w

        def idx_copy(s, g, kk):
            off = base + s * group + kk * chunk
            return pltpu.make_async_copy(
                idx_hbm.at[pl.ds(off, chunk)], idx_v.at[g * k_streams + kk], isem)

        def gather_copy(g, kk):
            slot = g * k_streams + kk
            return pltpu.make_async_copy(
                table_hbm.at[idx_v.at[slot]], rows_v.at[slot], gsem)

        def out_copy(s, g, kk):
            off = base + s * group + kk * chunk
            return pltpu.make_async_copy(
                rows_v.at[g * k_streams + kk], out_hbm.at[pl.ds(off, chunk)], osem)

        for kk in range(k_streams):
            idx_copy(0, 0, kk).start()

        def body(s, carry):
            g = lax.rem(s, 2)
            og = 1 - g
            for kk in range(k_streams):
                idx_copy(s, g, kk).wait()
            for kk in range(k_streams):
                gather_copy(g, kk).start()

            @pl.when(s >= 1)
            def _():
                for kk in range(k_streams):
                    out_copy(s - 1, og, kk).wait()

            @pl.when(s < n_super - 1)
            def _():
                for kk in range(k_streams):
                    idx_copy(s + 1, og, kk).start()

            for kk in range(k_streams):
                gather_copy(g, kk).wait()
            for kk in range(k_streams):
                out_copy(s, g, kk).start()
            return carry

        lax.fori_loop(0, n_super, body, 0)
        for kk in range(k_streams):
            out_copy(n_super - 1, lax.rem(n_super - 1, 2), kk).wait()

    return k(table, idx)


def kernel(x, table):
    b, h = x.shape
    _, d = table.shape
    idx = x.reshape(b * h).astype(jnp.int32)
    out = _gather_rows(table, idx, chunk=400, k_streams=4)
    return out.reshape(b, h, d)


# speedup vs baseline: 1.3562x; 1.3562x over previous
import functools

import jax
import jax.numpy as jnp
from jax import lax
from jax.experimental import pallas as pl
from jax.experimental.pallas import tpu as pltpu
from jax.experimental.pallas import tpu_sc as plsc

_NC = 2
_NS = 16
_NW = _NC * _NS


def _gather_transposed(table, x_t):
    v, d = table.shape
    h_len, batch = x_t.shape
    bpw = batch // _NW
    mesh = plsc.VectorSubcoreMesh(core_axis_name="c", subcore_axis_name="s")

    @functools.partial(
        pl.kernel,
        out_type=jax.ShapeDtypeStruct((h_len, d, batch), table.dtype),
        mesh=mesh,
        scratch_types=[
            pltpu.VMEM((2, bpw), jnp.int32),
            pltpu.VMEM((2, bpw, d), jnp.float32),
            pltpu.VMEM((2, d, bpw), jnp.float32),
            pltpu.SemaphoreType.DMA,
            pltpu.SemaphoreType.DMA,
            pltpu.SemaphoreType.DMA,
        ],
        compiler_params=pltpu.CompilerParams(
            use_tc_tiling_on_sc=False, needs_layout_passes=False),
    )
    def k(table_hbm, xt_hbm, out_hbm, idx_v, rows_v, trows_v, isem, gsem, osem):
        wid = lax.axis_index("s") * _NC + lax.axis_index("c")
        bbase = wid * bpw

        def idx_copy(h, slot):
            return pltpu.make_async_copy(
                xt_hbm.at[h, pl.ds(bbase, bpw)], idx_v.at[slot], isem)

        def gather_copy(slot):
            return pltpu.make_async_copy(
                table_hbm.at[idx_v.at[slot]], rows_v.at[slot], gsem)

        def out_copy(h, slot):
            return pltpu.make_async_copy(
                trows_v.at[slot], out_hbm.at[h, pl.ds(0, d), pl.ds(bbase, bpw)],
                osem)

        iota = lax.iota(jnp.int32, 16)

        idx_copy(0, 0).start()
        idx_copy(0, 0).wait()
        gather_copy(0).start()
        idx_copy(1, 1).start()

        def body(h, carry):
            cur = lax.rem(h, 2)
            nxt = 1 - cur
            gather_copy(cur).wait()

            @pl.when(h < h_len - 1)
            def _():
                idx_copy(h + 1, nxt).wait()
                gather_copy(nxt).start()

            @pl.when(h < h_len - 2)
            def _():
                idx_copy(h + 2, cur).start()

            @pl.when(h >= 2)
            def _():
                out_copy(h - 2, cur).wait()

            rows = rows_v.at[cur]

            def tbody(g, carry2):
                riv = g * 16 + iota
                for dd in range(d):
                    col = plsc.load_gather(
                        rows, [riv, jnp.full((16,), dd, jnp.int32)])
                    trows_v[cur, dd, pl.ds(g * 16, 16)] = col
                return carry2

            lax.fori_loop(0, bpw // 16, tbody, 0)
            out_copy(h, cur).start()
            return carry

        lax.fori_loop(0, h_len, body, 0)
        out_copy(h_len - 2, 0).wait()
        out_copy(h_len - 1, 1).wait()

    return k(table, x_t)


def kernel(x, table):
    x_t = x.T.astype(jnp.int32)
    out_t = _gather_transposed(table, x_t)
    return jnp.transpose(out_t, (2, 0, 1))


# speedup vs baseline: 1.6754x; 1.2354x over previous
import functools

import jax
import jax.numpy as jnp
from jax import lax
from jax.experimental import pallas as pl
from jax.experimental.pallas import tpu as pltpu
from jax.experimental.pallas import tpu_sc as plsc

_NC = 2
_NS = 16
_NW = _NC * _NS


def _gather_transposed(table, x_t):
    v, d = table.shape
    h_len, batch = x_t.shape
    bpw = batch // _NW
    mesh = plsc.VectorSubcoreMesh(core_axis_name="c", subcore_axis_name="s")

    @functools.partial(
        pl.kernel,
        out_type=jax.ShapeDtypeStruct((h_len, d, batch), table.dtype),
        mesh=mesh,
        scratch_types=[
            pltpu.VMEM((2, bpw), jnp.int32),
            pltpu.VMEM((2, bpw, d), jnp.float32),
            pltpu.VMEM((2, d, bpw), jnp.float32),
            pltpu.SemaphoreType.DMA,
            pltpu.SemaphoreType.DMA,
            pltpu.SemaphoreType.DMA,
        ],
        compiler_params=pltpu.CompilerParams(
            use_tc_tiling_on_sc=False, needs_layout_passes=False,
            disable_bounds_checks=True),
    )
    def k(table_hbm, xt_hbm, out_hbm, idx_v, rows_v, trows_v, isem, gsem, osem):
        wid = lax.axis_index("s") * _NC + lax.axis_index("c")
        bbase = wid * bpw

        def idx_copy(h, slot):
            return pltpu.make_async_copy(
                xt_hbm.at[h, pl.ds(bbase, bpw)], idx_v.at[slot], isem)

        def gather_copy(slot):
            return pltpu.make_async_copy(
                table_hbm.at[idx_v.at[slot]], rows_v.at[slot], gsem)

        def out_copy(h, slot):
            return pltpu.make_async_copy(
                trows_v.at[slot], out_hbm.at[h, pl.ds(0, d), pl.ds(bbase, bpw)],
                osem)

        iota = lax.iota(jnp.int32, 16)
        dconsts = [jnp.full((16,), dd, jnp.int32) for dd in range(d)]

        idx_copy(0, 0).start()
        idx_copy(0, 0).wait()
        gather_copy(0).start()
        idx_copy(1, 1).start()

        def body(h, carry):
            cur = lax.rem(h, 2)
            nxt = 1 - cur
            gather_copy(cur).wait()

            @pl.when(h < h_len - 1)
            def _():
                idx_copy(h + 1, nxt).wait()
                gather_copy(nxt).start()

            @pl.when(h < h_len - 2)
            def _():
                idx_copy(h + 2, cur).start()

            @pl.when(h >= 2)
            def _():
                out_copy(h - 2, cur).wait()

            rows = rows_v.at[cur]

            @plsc.parallel_loop(0, bpw // 16, unroll=2)
            def tloop(g):
                riv = g * 16 + iota
                for dd in range(d):
                    col = plsc.load_gather(rows, [riv, dconsts[dd]])
                    trows_v[cur, dd, pl.ds(g * 16, 16)] = col
            out_copy(h, cur).start()
            return carry

        lax.fori_loop(0, h_len, body, 0)
        out_copy(h_len - 2, 0).wait()
        out_copy(h_len - 1, 1).wait()

    return k(table, x_t)


def kernel(x, table):
    x_t = x.T.astype(jnp.int32)
    out_t = _gather_transposed(table, x_t)
    return jnp.transpose(out_t, (2, 0, 1))
